# Initial kernel scaffold; baseline (speedup 1.0000x reference)
import jax
import jax.numpy as jnp
from jax.experimental import pallas as pl

N = 10000
E = 160000
HID = 64
H = 4
L = 4
G = 64


def _mlp_head(pooled_ref, W1_ref, b1_ref, W2_ref, b2_ref, W3_ref, b3_ref, out_ref):
    z = jnp.maximum(jnp.dot(pooled_ref[...], W1_ref[...]) + b1_ref[...], 0.0)
    z = jnp.maximum(jnp.dot(z, W2_ref[...]) + b2_ref[...], 0.0)
    z = jnp.dot(z, W3_ref[...]) + b3_ref[...]
    out_ref[...] = z


def kernel(x, edge_index, edge_attr, batch, Wp, bp, Wl, bl, Wr, br, We, att, bgat, ln_g, ln_b, Wres, bres, W1, b1, W2, b2, W3, b3):
    src = edge_index[0]
    dst = edge_index[1]
    h = jax.nn.elu(jnp.dot(x, Wp) + bp)
    for i in range(L):
        h_res = h
        xl = (jnp.dot(h, Wl[i]) + bl[i]).reshape(N, H, HID)
        xr = (jnp.dot(h, Wr[i]) + br[i]).reshape(N, H, HID)
        e = jnp.dot(edge_attr, We[i]).reshape(E, H, HID)
        m = xl[src] + xr[dst] + e
        m = jax.nn.leaky_relu(m, 0.2)
        alpha = (m * att[i][None, :, :]).sum(-1)
        ex = jnp.exp(alpha)
        denom = jax.ops.segment_sum(ex, dst, num_segments=N)
        msg = xl[src] * ex[:, :, None]
        out = jax.ops.segment_sum(msg, dst, num_segments=N)
        out = out / (denom[:, :, None] + 1e-16)
        out = out.mean(axis=1) + bgat[i]
        mu = out.mean(-1, keepdims=True)
        var = out.var(-1, keepdims=True)
        out = (out - mu) / jnp.sqrt(var + 1e-5) * ln_g[i] + ln_b[i]
        out = jax.nn.elu(out)
        if i > 0:
            out = out + (jnp.dot(h_res, Wres[i - 1]) + bres[i - 1])
        h = out
    sums = jax.ops.segment_sum(h, batch, num_segments=G)
    cnt = jax.ops.segment_sum(jnp.ones((N,), jnp.float32), batch, num_segments=G)
    pooled = sums / jnp.maximum(cnt, 1.0)[:, None]
    z = pl.pallas_call(
        _mlp_head,
        out_shape=jax.ShapeDtypeStruct((G, 1), jnp.float32),
    )(pooled, W1, b1, W2, b2, W3, b3)
    return z.reshape(-1)


# speedup vs baseline: 1.1117x; 1.1117x over previous
import jax
import jax.numpy as jnp
from jax.experimental import pallas as pl

N = 10000
E = 160000
HID = 64
H = 4
L = 4
G = 64


def _mlp_head(pooled_ref, W1_ref, b1_ref, W2_ref, b2_ref, W3_ref, b3_ref, out_ref):
    z = jnp.maximum(jnp.dot(pooled_ref[...], W1_ref[...]) + b1_ref[...], 0.0)
    z = jnp.maximum(jnp.dot(z, W2_ref[...]) + b2_ref[...], 0.0)
    z = jnp.dot(z, W3_ref[...]) + b3_ref[...]
    out_ref[...] = z


def kernel(x, edge_index, edge_attr, batch, Wp, bp, Wl, bl, Wr, br, We, att, bgat, ln_g, ln_b, Wres, bres, W1, b1, W2, b2, W3, b3):
    src = edge_index[0]
    dst = edge_index[1]
    h = jax.nn.elu(jnp.dot(x, Wp) + bp)
    for i in range(L):
        h_res = h
        xl = (jnp.dot(h, Wl[i]) + bl[i]).reshape(N, H, HID)
        xr = (jnp.dot(h, Wr[i]) + br[i]).reshape(N, H, HID)
        e = jnp.dot(edge_attr, We[i]).reshape(E, H, HID)
        m = xl[src] + xr[dst] + e
        m = jax.nn.leaky_relu(m, 0.2)
        alpha = (m * att[i][None, :, :]).sum(-1)
        ex = jnp.exp(alpha)
        denom = jax.ops.segment_sum(ex, dst, num_segments=N)
        msg = xl[src] * ex[:, :, None]
        out = jax.ops.segment_sum(msg, dst, num_segments=N)
        out = out / (denom[:, :, None] + 1e-16)
        out = out.mean(axis=1) + bgat[i]
        mu = out.mean(-1, keepdims=True)
        var = out.var(-1, keepdims=True)
        out = (out - mu) / jnp.sqrt(var + 1e-5) * ln_g[i] + ln_b[i]
        out = jax.nn.elu(out)
        if i > 0:
            out = out + (jnp.dot(h_res, Wres[i - 1]) + bres[i - 1])
        h = out
    sums = jax.ops.segment_sum(h, batch, num_segments=G)
    cnt = jax.ops.segment_sum(jnp.ones((N,), jnp.float32), batch, num_segments=G)
    pooled = sums / jnp.maximum(cnt, 1.0)[:, None]
    z = jnp.maximum(jnp.dot(pooled, W1) + b1, 0.0)
    z = jnp.maximum(jnp.dot(z, W2) + b2, 0.0)
    z = jnp.dot(z, W3) + b3
    return z.reshape(-1)


# speedup vs baseline: 11.6124x; 10.4452x over previous
import functools

import jax
import jax.numpy as jnp
from jax import lax
from jax.experimental import pallas as pl
from jax.experimental.pallas import tpu as pltpu
from jax.experimental.pallas import tpu_sc as plsc

N = 10000
E = 160000
D_IN = 128
D_EDGE = 16
HID = 64
H = 4
L = 4
G = 64

NSC = 2
NTEC = 16
EB = 64
NBLK = E // EB
NP = 10240
RPT = NP // NTEC


def _sc_edge_body(src_hbm, dst_hbm, dsto_hbm, xl_hbm, xr_hbm, e_hbm, att_hbm,
                  acc_hbm, d0_hbm, d1_hbm, sidx, didx, gidx, xlb, xrb, msgb,
                  den01, attv, acc_sh, sem):
    cid = lax.axis_index("c")
    sid = lax.axis_index("s")

    pltpu.sync_copy(att_hbm.at[pl.ds(cid * 128, 128)], attv)

    zero = jnp.zeros((16,), jnp.float32)

    def zden(r, _):
        den01[pl.ds(r * 16, 16)] = zero
        return 0

    lax.fori_loop(0, 2 * NP // 16, zden, 0)

    def zrow(r, _):
        for j in range(8):
            msgb[r, pl.ds(j * 16, 16)] = zero
        return 0

    lax.fori_loop(0, EB, zrow, 0)
    base_row = sid * RPT
    for k in range(RPT // EB):
        pltpu.sync_copy(msgb, acc_sh.at[pl.ds(base_row + k * EB, EB)])
    plsc.subcore_barrier()

    att_v = [attv[pl.ds(j * 16, 16)] for j in range(8)]
    iot = lax.iota(jnp.int32, 16)
    lanemask = [iot == r2 for r2 in range(16)]
    perms = [iot ^ (1 << p) for p in range(4)]
    gdn = lax.GatherDimensionNumbers(
        offset_dims=(), collapsed_slice_dims=(0,), start_index_map=(0,))

    def _allsum(v):
        for p in perms:
            v = v + lax.gather(v, p[:, None], gdn, (1,),
                               mode=lax.GatherScatterMode.PROMISE_IN_BOUNDS)
        return v

    nblk = (NBLK + NTEC - 1 - sid) // NTEC

    def blk(k, _):
        off = (sid + NTEC * k) * EB
        pltpu.sync_copy(src_hbm.at[pl.ds(cid * E + off, EB)], sidx)
        pltpu.sync_copy(dst_hbm.at[pl.ds(off, EB)], didx)
        pltpu.sync_copy(dsto_hbm.at[pl.ds(cid * E + off, EB)], gidx)
        pltpu.async_copy(xl_hbm.at[sidx], xlb, sem).wait()
        pltpu.async_copy(xr_hbm.at[gidx], xrb, sem).wait()
        pltpu.sync_copy(e_hbm.at[pl.ds(cid * E + off, EB)], msgb)

        def chunk(c, _):
            idxv = didx[pl.ds(c * 16, 16)]
            packed0 = zero
            packed1 = zero
            for r2 in range(16):
                r = c * 16 + r2
                xs = [xlb[r, pl.ds(j * 16, 16)] for j in range(8)]
                ts = []
                for j in range(8):
                    m = xs[j] + xrb[r, pl.ds(j * 16, 16)] + msgb[r, pl.ds(j * 16, 16)]
                    m = jnp.maximum(m, 0.2 * m)
                    ts.append(m * att_v[j])
                e0 = jnp.exp(_allsum(ts[0] + ts[1] + ts[2] + ts[3]))
                e1 = jnp.exp(_allsum(ts[4] + ts[5] + ts[6] + ts[7]))
                for j in range(4):
                    msgb[r, pl.ds(j * 16, 16)] = xs[j] * e0
                for j in range(4, 8):
                    msgb[r, pl.ds(j * 16, 16)] = xs[j] * e1
                packed0 = jnp.where(lanemask[r2], e0, packed0)
                packed1 = jnp.where(lanemask[r2], e1, packed1)
            plsc.addupdate_scatter(den01, [idxv], packed0)
            plsc.addupdate_scatter(den01, [idxv + NP], packed1)
            return 0

        lax.fori_loop(0, EB // 16, chunk, 0)
        pltpu.sync_copy(msgb, acc_sh.at[didx], add=True)
        return 0

    lax.fori_loop(0, nblk, blk, 0)

    wrow = cid * NTEC + sid
    pltpu.sync_copy(den01.at[pl.ds(0, NP)], d0_hbm.at[pl.ds(wrow * NP, NP)])
    pltpu.sync_copy(den01.at[pl.ds(NP, NP)], d1_hbm.at[pl.ds(wrow * NP, NP)])

    plsc.subcore_barrier()
    for k in range(RPT // EB):
        pltpu.sync_copy(acc_sh.at[pl.ds(base_row + k * EB, EB)],
                        acc_hbm.at[pl.ds(cid * NP + base_row + k * EB, EB)])


_sc_edge = pl.kernel(
    _sc_edge_body,
    out_type=[
        jax.ShapeDtypeStruct((NSC * NP, 128), jnp.float32),
        jax.ShapeDtypeStruct((NSC * NTEC * NP,), jnp.float32),
        jax.ShapeDtypeStruct((NSC * NTEC * NP,), jnp.float32),
    ],
    mesh=plsc.VectorSubcoreMesh(core_axis_name="c", subcore_axis_name="s"),
    compiler_params=pltpu.CompilerParams(needs_layout_passes=False),
    scratch_types=[
        pltpu.VMEM((EB,), jnp.int32),
        pltpu.VMEM((EB,), jnp.int32),
        pltpu.VMEM((EB,), jnp.int32),
        pltpu.VMEM((EB, 128), jnp.float32),
        pltpu.VMEM((EB, 128), jnp.float32),
        pltpu.VMEM((EB, 128), jnp.float32),
        pltpu.VMEM((2 * NP,), jnp.float32),
        pltpu.VMEM((128,), jnp.float32),
        pltpu.VMEM_SHARED((NP, 128), jnp.float32),
        pltpu.SemaphoreType.DMA,
    ],
)


def _denred_body(d0_ref, d1_ref, out_ref):
    out_ref[0:1] = jnp.sum(d0_ref[0:16], axis=0, keepdims=True)
    out_ref[1:2] = jnp.sum(d1_ref[0:16], axis=0, keepdims=True)
    out_ref[2:3] = jnp.sum(d0_ref[16:32], axis=0, keepdims=True)
    out_ref[3:4] = jnp.sum(d1_ref[16:32], axis=0, keepdims=True)


def _denred(d0p, d1p):
    return pl.pallas_call(
        _denred_body,
        grid=(NP // 512,),
        in_specs=[
            pl.BlockSpec((2 * NTEC, 512), lambda i: (0, i)),
            pl.BlockSpec((2 * NTEC, 512), lambda i: (0, i)),
        ],
        out_specs=pl.BlockSpec((4, 512), lambda i: (0, i)),
        out_shape=jax.ShapeDtypeStruct((4, NP), jnp.float32),
    )(d0p.reshape(2 * NTEC, NP), d1p.reshape(2 * NTEC, NP))


_RB = 400


def _pre_body(x_ref, wp_ref, bp_ref, wl_ref, bl_ref, wr_ref, br_ref,
              h_ref, xl_ref, xr_ref):
    h = jnp.dot(x_ref[...], wp_ref[...], preferred_element_type=jnp.float32)
    h = h + bp_ref[...]
    h = jnp.where(h > 0, h, jnp.exp(h) - 1.0)
    h_ref[...] = h
    xl = jnp.dot(h, wl_ref[...], preferred_element_type=jnp.float32) + bl_ref[...]
    xr = jnp.dot(h, wr_ref[...], preferred_element_type=jnp.float32) + br_ref[...]
    xl_ref[0] = xl[:, :128]
    xl_ref[1] = xl[:, 128:]
    xr_ref[0] = xr[:, :128]
    xr_ref[1] = xr[:, 128:]


def _pre(x, Wp, bp, Wl0, bl0, Wr0, br0):
    return pl.pallas_call(
        _pre_body,
        grid=(N // _RB,),
        in_specs=[
            pl.BlockSpec((_RB, D_IN), lambda i: (i, 0)),
            pl.BlockSpec((D_IN, HID), lambda i: (0, 0)),
            pl.BlockSpec((1, HID), lambda i: (0, 0)),
            pl.BlockSpec((HID, H * HID), lambda i: (0, 0)),
            pl.BlockSpec((1, H * HID), lambda i: (0, 0)),
            pl.BlockSpec((HID, H * HID), lambda i: (0, 0)),
            pl.BlockSpec((1, H * HID), lambda i: (0, 0)),
        ],
        out_specs=[
            pl.BlockSpec((_RB, HID), lambda i: (i, 0)),
            pl.BlockSpec((2, _RB, 128), lambda i: (0, i, 0)),
            pl.BlockSpec((2, _RB, 128), lambda i: (0, i, 0)),
        ],
        out_shape=[
            jax.ShapeDtypeStruct((N, HID), jnp.float32),
            jax.ShapeDtypeStruct((2, N, 128), jnp.float32),
            jax.ShapeDtypeStruct((2, N, 128), jnp.float32),
        ],
    )(x, Wp, bp.reshape(1, HID), Wl0, bl0.reshape(1, H * HID),
      Wr0, br0.reshape(1, H * HID))


_EBLK = 2000


def _e_body(ea_ref, we_ref, out_ref):
    e = jnp.dot(ea_ref[...], we_ref[0], preferred_element_type=jnp.float32)
    out_ref[0, 0] = e[:, :128]
    out_ref[0, 1] = e[:, 128:]


def _e_all(edge_attr, We):
    return pl.pallas_call(
        _e_body,
        grid=(L, E // _EBLK),
        in_specs=[
            pl.BlockSpec((_EBLK, D_EDGE), lambda l, i: (i, 0)),
            pl.BlockSpec((1, D_EDGE, H * HID), lambda l, i: (l, 0, 0)),
        ],
        out_specs=pl.BlockSpec((1, 2, _EBLK, 128), lambda l, i: (l, 0, i, 0)),
        out_shape=jax.ShapeDtypeStruct((L, 2, E, 128), jnp.float32),
    )(edge_attr, We)


def _post_body_gen(has_res, has_next):
    def body(*refs):
        i = 0
        acc_ref = refs[i]; i += 1
        den_ref = refs[i]; i += 1
        h_ref = refs[i]; i += 1
        bg_ref = refs[i]; i += 1
        g_ref = refs[i]; i += 1
        b_ref = refs[i]; i += 1
        if has_res:
            wres_ref = refs[i]; i += 1
            bres_ref = refs[i]; i += 1
        if has_next:
            wl_ref = refs[i]; i += 1
            bl_ref = refs[i]; i += 1
            wr_ref = refs[i]; i += 1
            br_ref = refs[i]; i += 1
        hout_ref = refs[i]; i += 1
        if has_next:
            xlo_ref = refs[i]; i += 1
            xro_ref = refs[i]; i += 1

        a0 = acc_ref[0]
        a1 = acc_ref[1]
        d = den_ref[...]
        eps = jnp.float32(1e-16)
        out = (a0[:, 0:64] / (d[:, 0:1] + eps)
               + a0[:, 64:128] / (d[:, 1:2] + eps)
               + a1[:, 0:64] / (d[:, 2:3] + eps)
               + a1[:, 64:128] / (d[:, 3:4] + eps)) * 0.25
        out = out + bg_ref[...]
        mu = jnp.mean(out, axis=1, keepdims=True)
        var = jnp.mean((out - mu) * (out - mu), axis=1, keepdims=True)
        out = (out - mu) * lax.rsqrt(var + 1e-5) * g_ref[...] + b_ref[...]
        out = jnp.where(out > 0, out, jnp.exp(out) - 1.0)
        if has_res:
            out = out + jnp.dot(h_ref[...], wres_ref[...],
                                preferred_element_type=jnp.float32) + bres_ref[...]
        hout_ref[...] = out
        if has_next:
            xl = jnp.dot(out, wl_ref[...], preferred_element_type=jnp.float32) + bl_ref[...]
            xr = jnp.dot(out, wr_ref[...], preferred_element_type=jnp.float32) + br_ref[...]
            xlo_ref[0] = xl[:, :128]
            xlo_ref[1] = xl[:, 128:]
            xro_ref[0] = xr[:, :128]
            xro_ref[1] = xr[:, 128:]
    return body


def _post(acc, den_t, h, bg, g, b, wres, bres, wl, bl, wr, br):
    has_res = wres is not None
    has_next = wl is not None
    in_specs = [
        pl.BlockSpec((2, _RB, 128), lambda i: (0, i, 0)),
        pl.BlockSpec((_RB, 4), lambda i: (i, 0)),
        pl.BlockSpec((_RB, HID), lambda i: (i, 0)),
        pl.BlockSpec((1, HID), lambda i: (0, 0)),
        pl.BlockSpec((1, HID), lambda i: (0, 0)),
        pl.BlockSpec((1, HID), lambda i: (0, 0)),
    ]
    args = [acc.reshape(2, NP, 128), den_t, h, bg.reshape(1, HID),
            g.reshape(1, HID), b.reshape(1, HID)]
    if has_res:
        in_specs += [pl.BlockSpec((HID, HID), lambda i: (0, 0)),
                     pl.BlockSpec((1, HID), lambda i: (0, 0))]
        args += [wres, bres.reshape(1, HID)]
    if has_next:
        in_specs += [pl.BlockSpec((HID, H * HID), lambda i: (0, 0)),
                     pl.BlockSpec((1, H * HID), lambda i: (0, 0)),
                     pl.BlockSpec((HID, H * HID), lambda i: (0, 0)),
                     pl.BlockSpec((1, H * HID), lambda i: (0, 0))]
        args += [wl, bl.reshape(1, H * HID), wr, br.reshape(1, H * HID)]
    out_specs = [pl.BlockSpec((_RB, HID), lambda i: (i, 0))]
    out_shape = [jax.ShapeDtypeStruct((N, HID), jnp.float32)]
    if has_next:
        out_specs += [pl.BlockSpec((2, _RB, 128), lambda i: (0, i, 0)),
                      pl.BlockSpec((2, _RB, 128), lambda i: (0, i, 0))]
        out_shape += [jax.ShapeDtypeStruct((2, N, 128), jnp.float32),
                      jax.ShapeDtypeStruct((2, N, 128), jnp.float32)]
    return pl.pallas_call(
        _post_body_gen(has_res, has_next),
        grid=(N // _RB,),
        in_specs=in_specs,
        out_specs=out_specs,
        out_shape=out_shape,
    )(*args)


def _final_body(h_ref, batch_ref, w1_ref, b1_ref, w2_ref, b2_ref, w3_ref,
                b3_ref, out_ref):
    h = h_ref[...]
    bvec = batch_ref[...]
    gids = lax.broadcasted_iota(jnp.int32, (G, N), 0)
    onehot = (bvec == gids).astype(jnp.float32)
    sums = jnp.dot(onehot, h, preferred_element_type=jnp.float32)
    cnt = jnp.sum(onehot, axis=1, keepdims=True)
    pooled = sums / jnp.maximum(cnt, 1.0)
    z = jnp.dot(pooled, w1_ref[...], preferred_element_type=jnp.float32) + b1_ref[...]
    z = jnp.maximum(z, 0.0)
    z = jnp.dot(z, w2_ref[...], preferred_element_type=jnp.float32) + b2_ref[...]
    z = jnp.maximum(z, 0.0)
    z = jnp.dot(z, w3_ref[...], preferred_element_type=jnp.float32) + b3_ref[...]
    out_ref[...] = z


def _final(h, batch, W1, b1, W2, b2, W3, b3):
    return pl.pallas_call(
        _final_body,
        out_shape=jax.ShapeDtypeStruct((G, 1), jnp.float32),
    )(h, batch.reshape(1, N), W1, b1.reshape(1, HID),
      W2, b2.reshape(1, HID // 2), W3, b3.reshape(1, 1))


def kernel(x, edge_index, edge_attr, batch, Wp, bp, Wl, bl, Wr, br, We, att,
           bgat, ln_g, ln_b, Wres, bres, W1, b1, W2, b2, W3, b3):
    src = edge_index[0]
    dst = edge_index[1]
    src_off = jnp.concatenate([src, src + N])
    dst_off = jnp.concatenate([dst, dst + N])

    h, xl2, xr2 = _pre(x, Wp, bp, Wl[0], bl[0], Wr[0], br[0])
    e_all = _e_all(edge_attr, We)

    for i in range(L):
        acc, d0p, d1p = _sc_edge(src_off, dst, dst_off,
                                 xl2.reshape(2 * N, 128), xr2.reshape(2 * N, 128),
                                 e_all[i].reshape(2 * E, 128),
                                 att[i].reshape(H * HID))
        den_t = _denred(d0p, d1p).T
        if i < L - 1:
            wres = Wres[i - 1] if i > 0 else None
            bres_i = bres[i - 1] if i > 0 else None
            h, xl2, xr2 = _post(acc, den_t, h, bgat[i], ln_g[i], ln_b[i],
                                wres, bres_i, Wl[i + 1], bl[i + 1],
                                Wr[i + 1], br[i + 1])
        else:
            (h,) = _post(acc, den_t, h, bgat[i], ln_g[i], ln_b[i],
                         Wres[i - 1], bres[i - 1], None, None, None, None)

    z = _final(h, batch, W1, b1, W2, b2, W3, b3)
    return z.reshape(-1)


# speedup vs baseline: 13.8476x; 1.1925x over previous
import functools

import jax
import jax.numpy as jnp
from jax import lax
from jax.experimental import pallas as pl
from jax.experimental.pallas import tpu as pltpu
from jax.experimental.pallas import tpu_sc as plsc

N = 10000
E = 160000
D_IN = 128
D_EDGE = 16
HID = 64
H = 4
L = 4
G = 64

NSC = 2
NTEC = 16
EB = 64
NBLK = E // EB
NP = 10240
RPT = NP // NTEC


def _sc_edge_body(src_hbm, dst_hbm, xl_hbm, xr_hbm, e_hbm, att_hbm,
                  acc_hbm, d0_hbm, d1_hbm, sidx, didx, gidx, xlb, xrb, msgb,
                  den01, attv, acc_sh, sem, sem2):
    cid = lax.axis_index("c")
    sid = lax.axis_index("s")

    pltpu.sync_copy(att_hbm.at[pl.ds(cid * 128, 128)], attv)

    zero = jnp.zeros((16,), jnp.float32)

    def zden(r, _):
        den01[pl.ds(r * 16, 16)] = zero
        return 0

    lax.fori_loop(0, 2 * NP // 16, zden, 0)

    def zrow(r, _):
        for j in range(8):
            msgb[r, pl.ds(j * 16, 16)] = zero
        return 0

    lax.fori_loop(0, EB, zrow, 0)
    base_row = sid * RPT
    for k in range(RPT // EB):
        pltpu.sync_copy(msgb, acc_sh.at[pl.ds(base_row + k * EB, EB)])
    plsc.subcore_barrier()

    att_v = [attv[pl.ds(j * 16, 16)] for j in range(8)]
    iot = lax.iota(jnp.int32, 16)
    lanemask = [iot == r2 for r2 in range(16)]
    perms = [iot ^ (1 << p) for p in range(4)]
    gdn = lax.GatherDimensionNumbers(
        offset_dims=(), collapsed_slice_dims=(0,), start_index_map=(0,))

    def _allsum(v):
        for p in perms:
            v = v + lax.gather(v, p[:, None], gdn, (1,),
                               mode=lax.GatherScatterMode.PROMISE_IN_BOUNDS)
        return v

    nblk = (NBLK + NTEC - 1 - sid) // NTEC

    def blk(k, _):
        off = (sid + NTEC * k) * EB
        pltpu.sync_copy(src_hbm.at[pl.ds(cid * E + off, EB)], sidx)
        pltpu.sync_copy(dst_hbm.at[pl.ds(off, EB)], didx)
        noff = cid * N
        for c in range(EB // 16):
            gidx[pl.ds(c * 16, 16)] = didx[pl.ds(c * 16, 16)] + noff
        c1 = pltpu.async_copy(xl_hbm.at[sidx], xlb, sem)
        c2 = pltpu.async_copy(xr_hbm.at[gidx], xrb, sem2)
        c3 = pltpu.async_copy(e_hbm.at[pl.ds(cid * E + off, EB)], msgb, sem)
        c1.wait()
        c2.wait()
        c3.wait()

        def chunk(c, _):
            idxv = didx[pl.ds(c * 16, 16)]
            packed0 = zero
            packed1 = zero
            for r2 in range(16):
                r = c * 16 + r2
                xs = [xlb[r, pl.ds(j * 16, 16)] for j in range(8)]
                ts = []
                for j in range(8):
                    m = xs[j] + xrb[r, pl.ds(j * 16, 16)] + msgb[r, pl.ds(j * 16, 16)]
                    m = jnp.maximum(m, 0.2 * m)
                    ts.append(m * att_v[j])
                e0 = jnp.exp(_allsum(ts[0] + ts[1] + ts[2] + ts[3]))
                e1 = jnp.exp(_allsum(ts[4] + ts[5] + ts[6] + ts[7]))
                for j in range(4):
                    msgb[r, pl.ds(j * 16, 16)] = xs[j] * e0
                for j in range(4, 8):
                    msgb[r, pl.ds(j * 16, 16)] = xs[j] * e1
                packed0 = jnp.where(lanemask[r2], e0, packed0)
                packed1 = jnp.where(lanemask[r2], e1, packed1)
            plsc.addupdate_scatter(den01, [idxv], packed0)
            plsc.addupdate_scatter(den01, [idxv + NP], packed1)
            return 0

        lax.fori_loop(0, EB // 16, chunk, 0)
        pltpu.sync_copy(msgb, acc_sh.at[didx], add=True)
        return 0

    lax.fori_loop(0, nblk, blk, 0)

    wrow = cid * NTEC + sid
    pltpu.sync_copy(den01.at[pl.ds(0, NP)], d0_hbm.at[pl.ds(wrow * NP, NP)])
    pltpu.sync_copy(den01.at[pl.ds(NP, NP)], d1_hbm.at[pl.ds(wrow * NP, NP)])

    plsc.subcore_barrier()
    for k in range(RPT // EB):
        pltpu.sync_copy(acc_sh.at[pl.ds(base_row + k * EB, EB)],
                        acc_hbm.at[pl.ds(cid * NP + base_row + k * EB, EB)])


_sc_edge = pl.kernel(
    _sc_edge_body,
    out_type=[
        jax.ShapeDtypeStruct((NSC * NP, 128), jnp.float32),
        jax.ShapeDtypeStruct((NSC * NTEC * NP,), jnp.float32),
        jax.ShapeDtypeStruct((NSC * NTEC * NP,), jnp.float32),
    ],
    mesh=plsc.VectorSubcoreMesh(core_axis_name="c", subcore_axis_name="s"),
    compiler_params=pltpu.CompilerParams(needs_layout_passes=False),
    scratch_types=[
        pltpu.VMEM((EB,), jnp.int32),
        pltpu.VMEM((EB,), jnp.int32),
        pltpu.VMEM((EB,), jnp.int32),
        pltpu.VMEM((EB, 128), jnp.float32),
        pltpu.VMEM((EB, 128), jnp.float32),
        pltpu.VMEM((EB, 128), jnp.float32),
        pltpu.VMEM((2 * NP,), jnp.float32),
        pltpu.VMEM((128,), jnp.float32),
        pltpu.VMEM_SHARED((NP, 128), jnp.float32),
        pltpu.SemaphoreType.DMA,
        pltpu.SemaphoreType.DMA,
    ],
)


def _denred_body(d0_ref, d1_ref, out_ref):
    out_ref[0:1] = jnp.sum(d0_ref[0:16], axis=0, keepdims=True)
    out_ref[1:2] = jnp.sum(d1_ref[0:16], axis=0, keepdims=True)
    out_ref[2:3] = jnp.sum(d0_ref[16:32], axis=0, keepdims=True)
    out_ref[3:4] = jnp.sum(d1_ref[16:32], axis=0, keepdims=True)


def _denred(d0p, d1p):
    return pl.pallas_call(
        _denred_body,
        grid=(NP // 512,),
        in_specs=[
            pl.BlockSpec((2 * NTEC, 512), lambda i: (0, i)),
            pl.BlockSpec((2 * NTEC, 512), lambda i: (0, i)),
        ],
        out_specs=pl.BlockSpec((4, 512), lambda i: (0, i)),
        out_shape=jax.ShapeDtypeStruct((4, NP), jnp.float32),
    )(d0p.reshape(2 * NTEC, NP), d1p.reshape(2 * NTEC, NP))


_RB = 400


def _pre_body(x_ref, wp_ref, bp_ref, wl_ref, bl_ref, wr_ref, br_ref,
              h_ref, xl_ref, xr_ref):
    h = jnp.dot(x_ref[...], wp_ref[...], preferred_element_type=jnp.float32)
    h = h + bp_ref[...]
    h = jnp.where(h > 0, h, jnp.exp(h) - 1.0)
    h_ref[...] = h
    xl = jnp.dot(h, wl_ref[...], preferred_element_type=jnp.float32) + bl_ref[...]
    xr = jnp.dot(h, wr_ref[...], preferred_element_type=jnp.float32) + br_ref[...]
    xl_ref[0] = xl[:, :128]
    xl_ref[1] = xl[:, 128:]
    xr_ref[0] = xr[:, :128]
    xr_ref[1] = xr[:, 128:]


def _pre(x, Wp, bp, Wl0, bl0, Wr0, br0):
    return pl.pallas_call(
        _pre_body,
        grid=(N // _RB,),
        in_specs=[
            pl.BlockSpec((_RB, D_IN), lambda i: (i, 0)),
            pl.BlockSpec((D_IN, HID), lambda i: (0, 0)),
            pl.BlockSpec((1, HID), lambda i: (0, 0)),
            pl.BlockSpec((HID, H * HID), lambda i: (0, 0)),
            pl.BlockSpec((1, H * HID), lambda i: (0, 0)),
            pl.BlockSpec((HID, H * HID), lambda i: (0, 0)),
            pl.BlockSpec((1, H * HID), lambda i: (0, 0)),
        ],
        out_specs=[
            pl.BlockSpec((_RB, HID), lambda i: (i, 0)),
            pl.BlockSpec((2, _RB, 128), lambda i: (0, i, 0)),
            pl.BlockSpec((2, _RB, 128), lambda i: (0, i, 0)),
        ],
        out_shape=[
            jax.ShapeDtypeStruct((N, HID), jnp.float32),
            jax.ShapeDtypeStruct((2, N, 128), jnp.float32),
            jax.ShapeDtypeStruct((2, N, 128), jnp.float32),
        ],
    )(x, Wp, bp.reshape(1, HID), Wl0, bl0.reshape(1, H * HID),
      Wr0, br0.reshape(1, H * HID))


_EBLK = 2000


def _e_body(ea_ref, we_ref, out_ref):
    e = jnp.dot(ea_ref[...], we_ref[0], preferred_element_type=jnp.float32)
    out_ref[0, 0] = e[:, :128]
    out_ref[0, 1] = e[:, 128:]


def _e_all(edge_attr, We):
    return pl.pallas_call(
        _e_body,
        grid=(L, E // _EBLK),
        in_specs=[
            pl.BlockSpec((_EBLK, D_EDGE), lambda l, i: (i, 0)),
            pl.BlockSpec((1, D_EDGE, H * HID), lambda l, i: (l, 0, 0)),
        ],
        out_specs=pl.BlockSpec((1, 2, _EBLK, 128), lambda l, i: (l, 0, i, 0)),
        out_shape=jax.ShapeDtypeStruct((L, 2, E, 128), jnp.float32),
    )(edge_attr, We)


def _post_body_gen(has_res, has_next):
    def body(*refs):
        i = 0
        acc_ref = refs[i]; i += 1
        den_ref = refs[i]; i += 1
        h_ref = refs[i]; i += 1
        bg_ref = refs[i]; i += 1
        g_ref = refs[i]; i += 1
        b_ref = refs[i]; i += 1
        if has_res:
            wres_ref = refs[i]; i += 1
            bres_ref = refs[i]; i += 1
        if has_next:
            wl_ref = refs[i]; i += 1
            bl_ref = refs[i]; i += 1
            wr_ref = refs[i]; i += 1
            br_ref = refs[i]; i += 1
        hout_ref = refs[i]; i += 1
        if has_next:
            xlo_ref = refs[i]; i += 1
            xro_ref = refs[i]; i += 1

        a0 = acc_ref[0]
        a1 = acc_ref[1]
        d = den_ref[...]
        eps = jnp.float32(1e-16)
        out = (a0[:, 0:64] / (d[:, 0:1] + eps)
               + a0[:, 64:128] / (d[:, 1:2] + eps)
               + a1[:, 0:64] / (d[:, 2:3] + eps)
               + a1[:, 64:128] / (d[:, 3:4] + eps)) * 0.25
        out = out + bg_ref[...]
        mu = jnp.mean(out, axis=1, keepdims=True)
        var = jnp.mean((out - mu) * (out - mu), axis=1, keepdims=True)
        out = (out - mu) * lax.rsqrt(var + 1e-5) * g_ref[...] + b_ref[...]
        out = jnp.where(out > 0, out, jnp.exp(out) - 1.0)
        if has_res:
            out = out + jnp.dot(h_ref[...], wres_ref[...],
                                preferred_element_type=jnp.float32) + bres_ref[...]
        hout_ref[...] = out
        if has_next:
            xl = jnp.dot(out, wl_ref[...], preferred_element_type=jnp.float32) + bl_ref[...]
            xr = jnp.dot(out, wr_ref[...], preferred_element_type=jnp.float32) + br_ref[...]
            xlo_ref[0] = xl[:, :128]
            xlo_ref[1] = xl[:, 128:]
            xro_ref[0] = xr[:, :128]
            xro_ref[1] = xr[:, 128:]
    return body


def _post(acc, den_t, h, bg, g, b, wres, bres, wl, bl, wr, br):
    has_res = wres is not None
    has_next = wl is not None
    in_specs = [
        pl.BlockSpec((2, _RB, 128), lambda i: (0, i, 0)),
        pl.BlockSpec((_RB, 4), lambda i: (i, 0)),
        pl.BlockSpec((_RB, HID), lambda i: (i, 0)),
        pl.BlockSpec((1, HID), lambda i: (0, 0)),
        pl.BlockSpec((1, HID), lambda i: (0, 0)),
        pl.BlockSpec((1, HID), lambda i: (0, 0)),
    ]
    args = [acc.reshape(2, NP, 128), den_t, h, bg.reshape(1, HID),
            g.reshape(1, HID), b.reshape(1, HID)]
    if has_res:
        in_specs += [pl.BlockSpec((HID, HID), lambda i: (0, 0)),
                     pl.BlockSpec((1, HID), lambda i: (0, 0))]
        args += [wres, bres.reshape(1, HID)]
    if has_next:
        in_specs += [pl.BlockSpec((HID, H * HID), lambda i: (0, 0)),
                     pl.BlockSpec((1, H * HID), lambda i: (0, 0)),
                     pl.BlockSpec((HID, H * HID), lambda i: (0, 0)),
                     pl.BlockSpec((1, H * HID), lambda i: (0, 0))]
        args += [wl, bl.reshape(1, H * HID), wr, br.reshape(1, H * HID)]
    out_specs = [pl.BlockSpec((_RB, HID), lambda i: (i, 0))]
    out_shape = [jax.ShapeDtypeStruct((N, HID), jnp.float32)]
    if has_next:
        out_specs += [pl.BlockSpec((2, _RB, 128), lambda i: (0, i, 0)),
                      pl.BlockSpec((2, _RB, 128), lambda i: (0, i, 0))]
        out_shape += [jax.ShapeDtypeStruct((2, N, 128), jnp.float32),
                      jax.ShapeDtypeStruct((2, N, 128), jnp.float32)]
    return pl.pallas_call(
        _post_body_gen(has_res, has_next),
        grid=(N // _RB,),
        in_specs=in_specs,
        out_specs=out_specs,
        out_shape=out_shape,
    )(*args)


def _final_body(h_ref, batch_ref, w1_ref, b1_ref, w2_ref, b2_ref, w3_ref,
                b3_ref, out_ref):
    h = h_ref[...]
    bvec = batch_ref[...]
    gids = lax.broadcasted_iota(jnp.int32, (G, N), 0)
    onehot = (bvec == gids).astype(jnp.float32)
    sums = jnp.dot(onehot, h, preferred_element_type=jnp.float32)
    cnt = jnp.sum(onehot, axis=1, keepdims=True)
    pooled = sums / jnp.maximum(cnt, 1.0)
    z = jnp.dot(pooled, w1_ref[...], preferred_element_type=jnp.float32) + b1_ref[...]
    z = jnp.maximum(z, 0.0)
    z = jnp.dot(z, w2_ref[...], preferred_element_type=jnp.float32) + b2_ref[...]
    z = jnp.maximum(z, 0.0)
    z = jnp.dot(z, w3_ref[...], preferred_element_type=jnp.float32) + b3_ref[...]
    out_ref[...] = z


def _final(h, batch, W1, b1, W2, b2, W3, b3):
    return pl.pallas_call(
        _final_body,
        out_shape=jax.ShapeDtypeStruct((G, 1), jnp.float32),
    )(h, batch.reshape(1, N), W1, b1.reshape(1, HID),
      W2, b2.reshape(1, HID // 2), W3, b3.reshape(1, 1))


def kernel(x, edge_index, edge_attr, batch, Wp, bp, Wl, bl, Wr, br, We, att,
           bgat, ln_g, ln_b, Wres, bres, W1, b1, W2, b2, W3, b3):
    src = edge_index[0]
    dst = edge_index[1]
    src_off = jnp.concatenate([src, src + N])

    h, xl2, xr2 = _pre(x, Wp, bp, Wl[0], bl[0], Wr[0], br[0])
    e_all = _e_all(edge_attr, We)

    for i in range(L):
        acc, d0p, d1p = _sc_edge(src_off, dst,
                                 xl2.reshape(2 * N, 128), xr2.reshape(2 * N, 128),
                                 e_all[i].reshape(2 * E, 128),
                                 att[i].reshape(H * HID))
        den_t = _denred(d0p, d1p).T
        if i < L - 1:
            wres = Wres[i - 1] if i > 0 else None
            bres_i = bres[i - 1] if i > 0 else None
            h, xl2, xr2 = _post(acc, den_t, h, bgat[i], ln_g[i], ln_b[i],
                                wres, bres_i, Wl[i + 1], bl[i + 1],
                                Wr[i + 1], br[i + 1])
        else:
            (h,) = _post(acc, den_t, h, bgat[i], ln_g[i], ln_b[i],
                         Wres[i - 1], bres[i - 1], None, None, None, None)

    z = _final(h, batch, W1, b1, W2, b2, W3, b3)
    return z.reshape(-1)
